# Initial kernel scaffold; baseline (speedup 1.0000x reference)
import jax
import jax.numpy as jnp
from jax.experimental import pallas as pl


def kernel(xa, xb):
    raise NotImplementedError("write your pallas kernel here")


# speedup vs baseline: 40.5683x; 40.5683x over previous
import functools

import jax
import jax.numpy as jnp
from jax.experimental import pallas as pl
from jax.experimental.pallas import tpu as pltpu


def _fused_kernel(xa_ref, xb_ref, o_ref, *, k, ch):
    xa = xa_ref[...]
    xb = xb_ref[...]
    c = 2 * ch

    ma = jnp.mean(xa, axis=1, keepdims=True)
    mb = jnp.mean(xb, axis=1, keepdims=True)
    m_sub = jnp.concatenate([ma, mb], axis=0)
    m_lane = jnp.transpose(jnp.broadcast_to(m_sub, (c, 128)))[0:1]

    sub_i = jax.lax.broadcasted_iota(jnp.int32, (c, c), 0)
    lane_i = jax.lax.broadcasted_iota(jnp.int32, (c, c), 1)
    gt = m_sub > m_lane
    eq = m_sub == m_lane
    before = jnp.logical_or(gt, jnp.logical_and(eq, sub_i < lane_i))
    rnk = jnp.sum(before.astype(jnp.int32), axis=0, keepdims=True)

    rows = jax.lax.broadcasted_iota(jnp.int32, (k, c), 0)
    p_sel = (jnp.minimum(rnk, k - 1) == rows).astype(jnp.float32)

    o_ref[...] = (
        jnp.dot(p_sel[:, :ch], xa, preferred_element_type=jnp.float32)
        + jnp.dot(p_sel[:, ch:], xb, preferred_element_type=jnp.float32))


def _concat_topk_fold(xa, xb, k):
    n, ch, h, w = xa.shape
    hw = h * w
    xa2 = xa.reshape(n, ch, hw)
    xb2 = xb.reshape(n, ch, hw)
    y = pl.pallas_call(
        functools.partial(_fused_kernel, k=k, ch=ch),
        out_shape=jax.ShapeDtypeStruct((n, k, hw), jnp.float32),
        grid=(n,),
        in_specs=[
            pl.BlockSpec((None, ch, hw), lambda i: (i, 0, 0)),
            pl.BlockSpec((None, ch, hw), lambda i: (i, 0, 0)),
        ],
        out_specs=pl.BlockSpec((None, k, hw), lambda i: (i, 0, 0)),
        compiler_params=pltpu.CompilerParams(
            dimension_semantics=("parallel",)),
    )(xa2, xb2)
    return y.reshape(n, k, h, w)


def kernel(xa, xb):
    return _concat_topk_fold(xa, xb, 128)


# speedup vs baseline: 49.2342x; 1.2136x over previous
import functools

import jax
import jax.numpy as jnp
from jax.experimental import pallas as pl
from jax.experimental.pallas import tpu as pltpu


def _fused_kernel(xa_ref, xb_ref, o_ref, *, k, ch, bps):
    c = 2 * ch
    sub_i = jax.lax.broadcasted_iota(jnp.int32, (c, c), 0)
    lane_i = jax.lax.broadcasted_iota(jnp.int32, (c, c), 1)
    rows = jax.lax.broadcasted_iota(jnp.int32, (k, c), 0)

    for e in range(bps):
        xa = xa_ref[e]
        xb = xb_ref[e]

        ma = jnp.mean(xa, axis=1, keepdims=True)
        mb = jnp.mean(xb, axis=1, keepdims=True)
        m_sub = jnp.concatenate([ma, mb], axis=0)
        m_lane = jnp.transpose(jnp.broadcast_to(m_sub, (c, 128)))[0:1]

        gt = m_sub > m_lane
        eq = m_sub == m_lane
        before = jnp.logical_or(gt, jnp.logical_and(eq, sub_i < lane_i))
        rnk = jnp.sum(before.astype(jnp.int32), axis=0, keepdims=True)

        p_sel = (jnp.minimum(rnk, k - 1) == rows).astype(jnp.float32)

        o_ref[e] = (
            jnp.dot(p_sel[:, :ch], xa, preferred_element_type=jnp.float32)
            + jnp.dot(p_sel[:, ch:], xb, preferred_element_type=jnp.float32))


def _concat_topk_fold(xa, xb, k, bps=1):
    n, ch, h, w = xa.shape
    hw = h * w
    xa2 = xa.reshape(n, ch, hw)
    xb2 = xb.reshape(n, ch, hw)
    y = pl.pallas_call(
        functools.partial(_fused_kernel, k=k, ch=ch, bps=bps),
        out_shape=jax.ShapeDtypeStruct((n, k, hw), jnp.float32),
        grid=(n // bps,),
        in_specs=[
            pl.BlockSpec((bps, ch, hw), lambda i: (i, 0, 0)),
            pl.BlockSpec((bps, ch, hw), lambda i: (i, 0, 0)),
        ],
        out_specs=pl.BlockSpec((bps, k, hw), lambda i: (i, 0, 0)),
        compiler_params=pltpu.CompilerParams(
            dimension_semantics=("parallel",)),
    )(xa2, xb2)
    return y.reshape(n, k, h, w)


def kernel(xa, xb):
    return _concat_topk_fold(xa, xb, 128, bps=4)


# speedup vs baseline: 50.4079x; 1.0238x over previous
import functools

import jax
import jax.numpy as jnp
from jax.experimental import pallas as pl
from jax.experimental.pallas import tpu as pltpu


def _fused_kernel(xa_ref, xb_ref, o_ref, *, k, ch, bps):
    c = 2 * ch
    sub_i = jax.lax.broadcasted_iota(jnp.int32, (c, c), 0)
    lane_i = jax.lax.broadcasted_iota(jnp.int32, (c, c), 1)
    rows = jax.lax.broadcasted_iota(jnp.int32, (k, c), 0)

    for e in range(bps):
        xa = xa_ref[e]
        xb = xb_ref[e]

        ma = jnp.mean(xa, axis=1, keepdims=True)
        mb = jnp.mean(xb, axis=1, keepdims=True)
        m_sub = jnp.concatenate([ma, mb], axis=0)
        m_lane = jnp.transpose(jnp.broadcast_to(m_sub, (c, 128)))[0:1]

        gt = m_sub > m_lane
        eq = m_sub == m_lane
        before = jnp.logical_or(gt, jnp.logical_and(eq, sub_i < lane_i))
        rnk = jnp.sum(before.astype(jnp.int32), axis=0, keepdims=True)

        p_sel = (jnp.minimum(rnk, k - 1) == rows).astype(jnp.float32)

        o_ref[e] = (
            jnp.dot(p_sel[:, :ch], xa, preferred_element_type=jnp.float32)
            + jnp.dot(p_sel[:, ch:], xb, preferred_element_type=jnp.float32))


def _concat_topk_fold(xa, xb, k, bps=1):
    n, ch, h, w = xa.shape
    hw = h * w
    xa2 = xa.reshape(n, ch, hw)
    xb2 = xb.reshape(n, ch, hw)
    y = pl.pallas_call(
        functools.partial(_fused_kernel, k=k, ch=ch, bps=bps),
        out_shape=jax.ShapeDtypeStruct((n, k, hw), jnp.float32),
        grid=(n // bps,),
        in_specs=[
            pl.BlockSpec((bps, ch, hw), lambda i: (i, 0, 0)),
            pl.BlockSpec((bps, ch, hw), lambda i: (i, 0, 0)),
        ],
        out_specs=pl.BlockSpec((bps, k, hw), lambda i: (i, 0, 0)),
        compiler_params=pltpu.CompilerParams(
            dimension_semantics=("parallel",)),
    )(xa2, xb2)
    return y.reshape(n, k, h, w)


def kernel(xa, xb):
    return _concat_topk_fold(xa, xb, 128, bps=8)
